# baseline (speedup 1.0000x reference)
import functools

import jax
import jax.numpy as jnp
from jax import lax
from jax.experimental import pallas as pl
from jax.experimental.pallas import tpu as pltpu
from jax.experimental.pallas import tpu_sc as plsc

_B = 4096
_D = 64
_H = 20
_L = 16
_GIDX = _H * _L

_info = plsc.get_sparse_core_info()
_NC = _info.num_cores
_NS = _info.num_subcores
_NW = _NC * _NS
_NGROUP = _B // _L
_GPW = _NGROUP // _NW


@functools.partial(
    pl.kernel,
    mesh=plsc.VectorSubcoreMesh(core_axis_name="c", subcore_axis_name="s"),
    out_type=jax.ShapeDtypeStruct((_B,), jnp.float32),
    compiler_params=pltpu.CompilerParams(
        needs_layout_passes=False, use_tc_tiling_on_sc=False),
    scratch_types=[
        pltpu.VMEM((_GIDX,), jnp.int32),
        pltpu.VMEM((_GIDX,), jnp.int32),
        pltpu.VMEM((_L,), jnp.int32),
        pltpu.VMEM((_GIDX, _D), jnp.float32),
        pltpu.VMEM((_GIDX, _D), jnp.float32),
        pltpu.VMEM((_L, _D), jnp.float32),
        pltpu.VMEM((_L,), jnp.float32),
        pltpu.SemaphoreType.DMA,
        pltpu.SemaphoreType.DMA,
        pltpu.SemaphoreType.DMA,
    ],
)
def _distmult_sc(s_hbm, o_hbm, p_hbm, table_hbm, out_hbm,
                 s_idx, o_idx, p_idx, s_rows, o_rows, p_rows,
                 out_buf, sem_s, sem_o, sem_p):
    wid = lax.axis_index("s") * _NC + lax.axis_index("c")
    zero = jnp.zeros((_L,), jnp.float32)
    lane = lax.iota(jnp.int32, _L)

    def group_body(j, carry):
        g = wid * _GPW + j
        pltpu.sync_copy(s_hbm.at[pl.ds(g * _GIDX, _GIDX)], s_idx)
        pltpu.sync_copy(o_hbm.at[pl.ds(g * _GIDX, _GIDX)], o_idx)
        pltpu.sync_copy(p_hbm.at[pl.ds(g * _L, _L)], p_idx)
        cs = pltpu.async_copy(table_hbm.at[s_idx], s_rows, sem_s)
        co = pltpu.async_copy(table_hbm.at[o_idx], o_rows, sem_o)
        cp = pltpu.async_copy(table_hbm.at[p_idx], p_rows, sem_p)
        cs.wait()
        co.wait()
        cp.wait()

        def f_body(h, c):
            fs, fo = c
            fs = fs + jnp.where(s_idx[pl.ds(h * _L, _L)] != 0, 1.0, 0.0)
            fo = fo + jnp.where(o_idx[pl.ds(h * _L, _L)] != 0, 1.0, 0.0)
            return fs, fo

        fs, fo = lax.fori_loop(0, _H, f_body, (zero, zero))
        scale = fs * fo

        def e_body(e, dots):
            def h_body(h, acc):
                a = list(acc)
                r = h * _L + e
                for gd in range(4):
                    sl = pl.ds(gd * _L, _L)
                    a[gd] = a[gd] + s_rows[r, sl]
                    a[4 + gd] = a[4 + gd] + o_rows[r, sl]
                return tuple(a)

            acc = lax.fori_loop(0, _H, h_body, (zero,) * 8)
            v = zero
            for gd in range(4):
                sl = pl.ds(gd * _L, _L)
                v = v + acc[gd] * p_rows[e, sl] * acc[4 + gd]
            return jnp.where(lane == e, jnp.sum(v), dots)

        dots = lax.fori_loop(0, _L, e_body, zero)

        x = scale * dots
        out_buf[...] = 1.0 / (1.0 + jnp.exp(-x))
        pltpu.sync_copy(out_buf, out_hbm.at[pl.ds(g * _L, _L)])
        return carry

    lax.fori_loop(0, _GPW, group_body, 0)


def kernel(s, o, p, table):
    s_flat = jnp.swapaxes(
        s.astype(jnp.int32).reshape(_NGROUP, _L, _H), 1, 2).reshape(-1)
    o_flat = jnp.swapaxes(
        o.astype(jnp.int32).reshape(_NGROUP, _L, _H), 1, 2).reshape(-1)
    p_flat = p.astype(jnp.int32).reshape(-1)
    return _distmult_sc(s_flat, o_flat, p_flat, table)
